# Initial kernel scaffold; baseline (speedup 1.0000x reference)
import jax
import jax.numpy as jnp
from jax.experimental import pallas as pl


def kernel(x, W_in, b_in, W_p, b_p, W_out, b_out, ptr_dest, jg_W, jg_b, ctx_strength, pointer_inits):
    raise NotImplementedError("write your pallas kernel here")


# speedup vs baseline: 2.8119x; 2.8119x over previous
import functools

import jax
import jax.numpy as jnp
from jax.experimental import pallas as pl
from jax.experimental.pallas import tpu as pltpu

_M = 2048
_D = 64
_NB = 4
_K = 2
_W = 2 * _K + 1
_TEMP = 8.0
_T = 16
_B = 256
_BBLK = 32
_HALF = _M / 2.0


def _body(x_ref, Win_ref, bin_ref, Wp_ref, bp_ref, Wout_ref, bout_ref,
          ptrdest_ref, jgW_ref, jgb_ref, ctx_ref, ptr0_ref,
          out_ref,
          mem_ref, hid_ref, ptr_ref, basei_ref, grows_ref, upd_ref, dest_ref):
    mem_ref[...] = jnp.zeros_like(mem_ref)
    hid_ref[...] = jnp.zeros_like(hid_ref)
    ptr_ref[...] = ptr0_ref[...]

    def step(t, carry):
        xt = x_ref[pl.ds(t, 1), :, :].reshape(_BBLK, 8)
        emb = jnp.dot(xt, Win_ref[...], preferred_element_type=jnp.float32) + bin_ref[...]
        h_acc = jnp.zeros((_BBLK, _D), dtype=jnp.float32)
        for bi in range(_NB):
            ptr = ptr_ref[bi]
            base_f = jnp.clip(jnp.floor(ptr), 0.0, _M - 1.0)
            basei_ref[...] = base_f.astype(jnp.int32)

            io = jax.lax.broadcasted_iota(jnp.int32, (_BBLK, _W), 1).astype(jnp.float32)
            idx_f = jnp.mod(base_f + (io - _K), float(_M))
            delta = jnp.mod(idx_f - ptr + _HALF, float(_M)) - _HALF
            z = -(delta * delta) / _TEMP
            z = z - jnp.max(z, axis=1, keepdims=True)
            e = jnp.exp(z)
            w = e / jnp.sum(e, axis=1, keepdims=True)

            def gather(b, _):
                base_b = basei_ref[b, 0]
                for o in range(_W):
                    idx = base_b + (o - _K)
                    idx = jnp.where(idx < 0, idx + _M, idx)
                    idx = jnp.where(idx >= _M, idx - _M, idx)
                    grows_ref[pl.ds(b, 1), pl.ds(o, 1), :] = (
                        mem_ref[pl.ds(b, 1), pl.ds(idx, 1), :].reshape(1, 1, _D))
                dest_ref[pl.ds(b, 1), :] = (
                    ptrdest_ref[bi, pl.ds(base_b, 1), :].reshape(1, 1))
                return 0

            jax.lax.fori_loop(0, _BBLK, gather, 0, unroll=4)

            grows = grows_ref[...]
            mem_read = jnp.zeros((_BBLK, _D), dtype=jnp.float32)
            for o in range(_W):
                mem_read = mem_read + w[:, o:o + 1] * grows[:, o, :]

            ctx = ctx_ref[bi]
            h1 = jnp.tanh(emb + ctx * mem_read + hid_ref[bi])
            h2 = jnp.dot(h1, Wp_ref[...], preferred_element_type=jnp.float32) + bp_ref[...]
            h2 = jnp.maximum(h2, 0.0)

            jl = jnp.sum(h2 * jgW_ref[bi, :][None, :], axis=1, keepdims=True) + jgb_ref[bi]
            p = 1.0 / (1.0 + jnp.exp(-jl))
            hard = (p > 0.5).astype(jnp.float32)
            jump = hard - p + p
            walk = jnp.mod(ptr + 1.0, float(_M))
            ptr_ref[bi] = jump * dest_ref[...] + (1.0 - jump) * walk
            hid_ref[bi] = h2

            for o in range(_W):
                upd_ref[:, pl.ds(o, 1), :] = (w[:, o:o + 1] * h2)[:, None, :]

            def scatter(b, _):
                base_b = basei_ref[b, 0]
                for o in range(_W):
                    idx = base_b + (o - _K)
                    idx = jnp.where(idx < 0, idx + _M, idx)
                    idx = jnp.where(idx >= _M, idx - _M, idx)
                    mem_ref[pl.ds(b, 1), pl.ds(idx, 1), :] += (
                        upd_ref[pl.ds(b, 1), pl.ds(o, 1), :])
                return 0

            jax.lax.fori_loop(0, _BBLK, scatter, 0, unroll=4)
            h_acc = h_acc + h2

        out_t = jnp.dot(h_acc * (1.0 / _NB), Wout_ref[...],
                        preferred_element_type=jnp.float32) + bout_ref[...]
        out_ref[:, pl.ds(t, 1), :] = out_t[:, None, :]
        return carry

    jax.lax.fori_loop(0, _T, step, 0)


@jax.jit
def kernel(x, W_in, b_in, W_p, b_p, W_out, b_out, ptr_dest, jg_W, jg_b,
           ctx_strength, pointer_inits):
    xr = jnp.swapaxes(x, 0, 1)
    grid = (_B // _BBLK,)
    out = pl.pallas_call(
        _body,
        grid=grid,
        in_specs=[
            pl.BlockSpec((_T, _BBLK, 8), lambda i: (0, i, 0)),
            pl.BlockSpec((8, _D), lambda i: (0, 0)),
            pl.BlockSpec((1, _D), lambda i: (0, 0)),
            pl.BlockSpec((_D, _D), lambda i: (0, 0)),
            pl.BlockSpec((1, _D), lambda i: (0, 0)),
            pl.BlockSpec((_D, 8), lambda i: (0, 0)),
            pl.BlockSpec((1, 8), lambda i: (0, 0)),
            pl.BlockSpec((_NB, _M, 1), lambda i: (0, 0, 0)),
            pl.BlockSpec((_NB, _D), lambda i: (0, 0)),
            pl.BlockSpec(memory_space=pltpu.SMEM),
            pl.BlockSpec(memory_space=pltpu.SMEM),
            pl.BlockSpec((_NB, _BBLK, 1), lambda i: (0, i, 0)),
        ],
        out_specs=pl.BlockSpec((_BBLK, _T, 8), lambda i: (i, 0, 0)),
        out_shape=jax.ShapeDtypeStruct((_B, _T, 8), jnp.float32),
        scratch_shapes=[
            pltpu.VMEM((_BBLK, _M, _D), jnp.float32),
            pltpu.VMEM((_NB, _BBLK, _D), jnp.float32),
            pltpu.VMEM((_NB, _BBLK, 1), jnp.float32),
            pltpu.VMEM((_BBLK, 1), jnp.int32),
            pltpu.VMEM((_BBLK, _W, _D), jnp.float32),
            pltpu.VMEM((_BBLK, _W, _D), jnp.float32),
            pltpu.VMEM((_BBLK, 1), jnp.float32),
        ],
    )(xr, W_in, b_in.reshape(1, _D), W_p, b_p.reshape(1, _D), W_out,
      b_out.reshape(1, 8), ptr_dest[..., None], jg_W, jg_b, ctx_strength,
      pointer_inits[..., None])
    return out


# speedup vs baseline: 2.9386x; 1.0451x over previous
import functools

import jax
import jax.numpy as jnp
from jax.experimental import pallas as pl
from jax.experimental.pallas import tpu as pltpu

_M = 2048
_D = 64
_NB = 4
_K = 2
_W = 2 * _K + 1
_WP = 8
_TEMP = 8.0
_T = 16
_B = 256
_BBLK = 32
_MP = _M + 24
_ROFF = 8
_HALF = _M / 2.0


def _body(x_ref, Win_ref, bin_ref, Wp_ref, bp_ref, Wout_ref, bout_ref,
          ptrdest_ref, jgW_ref, jgb_ref, ctx_ref, ptr0_ref,
          out_ref,
          mem_ref, hid_ref, ptr_ref, basei_ref, grows_ref, upd_ref, dest_ref):
    mem_ref[...] = jnp.zeros_like(mem_ref)
    hid_ref[...] = jnp.zeros_like(hid_ref)
    upd_ref[...] = jnp.zeros_like(upd_ref)
    ptr_ref[...] = ptr0_ref[...]

    def step(t, carry):
        xt = x_ref[pl.ds(t, 1), :, :].reshape(_BBLK, 8)
        emb = jnp.dot(xt, Win_ref[...], preferred_element_type=jnp.float32) + bin_ref[...]
        h_acc = jnp.zeros((_BBLK, _D), dtype=jnp.float32)
        for bi in range(_NB):
            ptr = ptr_ref[bi]
            base_f = jnp.clip(jnp.floor(ptr), 0.0, _M - 1.0)
            basei_ref[...] = base_f.astype(jnp.int32)

            io = jax.lax.broadcasted_iota(jnp.int32, (_BBLK, _W), 1).astype(jnp.float32)
            idx_f = jnp.mod(base_f + (io - _K), float(_M))
            delta = jnp.mod(idx_f - ptr + _HALF, float(_M)) - _HALF
            z = -(delta * delta) / _TEMP
            z = z - jnp.max(z, axis=1, keepdims=True)
            e = jnp.exp(z)
            w = e / jnp.sum(e, axis=1, keepdims=True)

            def gather(b, _):
                base_b = basei_ref[b, 0]
                grows_ref[pl.ds(b, 1), :, :] = (
                    mem_ref[pl.ds(b, 1), pl.ds(base_b + (_ROFF - _K), _WP), :])
                dest_ref[pl.ds(b, 1), :] = (
                    ptrdest_ref[bi, pl.ds(base_b, 1), :].reshape(1, 1))
                return 0

            jax.lax.fori_loop(0, _BBLK, gather, 0, unroll=8)

            grows = grows_ref[...]
            mem_read = jnp.zeros((_BBLK, _D), dtype=jnp.float32)
            for o in range(_W):
                mem_read = mem_read + w[:, o:o + 1] * grows[:, o, :]

            ctx = ctx_ref[bi]
            h1 = jnp.tanh(emb + ctx * mem_read + hid_ref[bi])
            h2 = jnp.dot(h1, Wp_ref[...], preferred_element_type=jnp.float32) + bp_ref[...]
            h2 = jnp.maximum(h2, 0.0)

            jl = jnp.sum(h2 * jgW_ref[bi, :][None, :], axis=1, keepdims=True) + jgb_ref[bi]
            p = 1.0 / (1.0 + jnp.exp(-jl))
            hard = (p > 0.5).astype(jnp.float32)
            jump = hard - p + p
            walk = jnp.mod(ptr + 1.0, float(_M))
            ptr_ref[bi] = jump * dest_ref[...] + (1.0 - jump) * walk
            hid_ref[bi] = h2

            for o in range(_W):
                upd_ref[:, pl.ds(o, 1), :] = (w[:, o:o + 1] * h2)[:, None, :]

            def scatter(b, _):
                base_b = basei_ref[b, 0]
                u = upd_ref[pl.ds(b, 1), :, :]
                mem_ref[pl.ds(b, 1), pl.ds(base_b + (_ROFF - _K), _WP), :] += u
                wrap_lo = base_b <= 3
                wrap_hi = base_b >= _M - 4
                start2 = jnp.where(wrap_lo, base_b + (_ROFF - _K) + _M,
                                   base_b + (_ROFF - _K) - _M)

                @pl.when(jnp.logical_or(wrap_lo, wrap_hi))
                def _():
                    mem_ref[pl.ds(b, 1), pl.ds(start2, _WP), :] += u

                return 0

            jax.lax.fori_loop(0, _BBLK, scatter, 0, unroll=8)
            h_acc = h_acc + h2

        out_t = jnp.dot(h_acc * (1.0 / _NB), Wout_ref[...],
                        preferred_element_type=jnp.float32) + bout_ref[...]
        out_ref[:, pl.ds(t, 1), :] = out_t[:, None, :]
        return carry

    jax.lax.fori_loop(0, _T, step, 0)


@jax.jit
def kernel(x, W_in, b_in, W_p, b_p, W_out, b_out, ptr_dest, jg_W, jg_b,
           ctx_strength, pointer_inits):
    xr = jnp.swapaxes(x, 0, 1)
    grid = (_B // _BBLK,)
    out = pl.pallas_call(
        _body,
        grid=grid,
        in_specs=[
            pl.BlockSpec((_T, _BBLK, 8), lambda i: (0, i, 0)),
            pl.BlockSpec((8, _D), lambda i: (0, 0)),
            pl.BlockSpec((1, _D), lambda i: (0, 0)),
            pl.BlockSpec((_D, _D), lambda i: (0, 0)),
            pl.BlockSpec((1, _D), lambda i: (0, 0)),
            pl.BlockSpec((_D, 8), lambda i: (0, 0)),
            pl.BlockSpec((1, 8), lambda i: (0, 0)),
            pl.BlockSpec((_NB, _M, 1), lambda i: (0, 0, 0)),
            pl.BlockSpec((_NB, _D), lambda i: (0, 0)),
            pl.BlockSpec(memory_space=pltpu.SMEM),
            pl.BlockSpec(memory_space=pltpu.SMEM),
            pl.BlockSpec((_NB, _BBLK, 1), lambda i: (0, i, 0)),
        ],
        out_specs=pl.BlockSpec((_BBLK, _T, 8), lambda i: (i, 0, 0)),
        out_shape=jax.ShapeDtypeStruct((_B, _T, 8), jnp.float32),
        scratch_shapes=[
            pltpu.VMEM((_BBLK, _MP, _D), jnp.float32),
            pltpu.VMEM((_NB, _BBLK, _D), jnp.float32),
            pltpu.VMEM((_NB, _BBLK, 1), jnp.float32),
            pltpu.VMEM((_BBLK, 1), jnp.int32),
            pltpu.VMEM((_BBLK, _WP, _D), jnp.float32),
            pltpu.VMEM((_BBLK, _WP, _D), jnp.float32),
            pltpu.VMEM((_BBLK, 1), jnp.float32),
        ],
        compiler_params=pltpu.CompilerParams(
            dimension_semantics=("arbitrary",),
            disable_bounds_checks=True,
        ),
    )(xr, W_in, b_in.reshape(1, _D), W_p, b_p.reshape(1, _D), W_out,
      b_out.reshape(1, 8), ptr_dest[..., None], jg_W, jg_b, ctx_strength,
      pointer_inits[..., None])
    return out


# speedup vs baseline: 3.4162x; 1.1625x over previous
import functools

import jax
import jax.numpy as jnp
from jax.experimental import pallas as pl
from jax.experimental.pallas import tpu as pltpu

_M = 2048
_D = 64
_NB = 4
_K = 2
_W = 2 * _K + 1
_WP = 8
_TEMP = 8.0
_T = 16
_B = 256
_BBLK = 32
_MP = _M + 24
_ROFF = 8
_HALF = _M / 2.0


def _body(x_ref, Win_ref, bin_ref, Wp_ref, bp_ref, Wout_ref, bout_ref,
          ptrdest_ref, jgW_ref, jgb_ref, ctx_ref, ptr0_ref,
          out_ref,
          mem_ref, hid_ref, ptr_ref, basei_a, basei_b, grows_ref, upd_ref,
          dest_ref, w_ref):
    mem_ref[...] = jnp.zeros_like(mem_ref)
    hid_ref[...] = jnp.zeros_like(hid_ref)
    upd_ref[...] = jnp.zeros_like(upd_ref)
    grows_ref[...] = jnp.zeros_like(grows_ref)
    ptr_ref[...] = ptr0_ref[...]
    bufs = (basei_a, basei_b)

    def window_w(ptr):
        base_f = jnp.clip(jnp.floor(ptr), 0.0, _M - 1.0)
        io = jax.lax.broadcasted_iota(jnp.int32, (_BBLK, _WP), 1).astype(jnp.float32)
        idx_f = jnp.mod(base_f + (io - _K), float(_M))
        delta = jnp.mod(idx_f - ptr + _HALF, float(_M)) - _HALF
        z = -(delta * delta) / _TEMP
        live = io < float(_W)
        z = jnp.where(live, z, -jnp.inf)
        z = z - jnp.max(z, axis=1, keepdims=True)
        e = jnp.exp(z)
        e = jnp.where(live, e, 0.0)
        return base_f, e / jnp.sum(e, axis=1, keepdims=True)

    base_f0, w0 = window_w(ptr_ref[0])
    w_ref[...] = w0
    basei_a[...] = base_f0.astype(jnp.int32)

    def dest0(b, _):
        base_b = basei_a[b, 0]
        dest_ref[pl.ds(b, 1), :] = ptrdest_ref[0, pl.ds(base_b, 1), :].reshape(1, 1)
        return 0

    jax.lax.fori_loop(0, _BBLK, dest0, 0, unroll=8)

    def step(t, carry):
        xt = x_ref[pl.ds(t, 1), :, :].reshape(_BBLK, 8)
        emb = jnp.dot(xt, Win_ref[...], preferred_element_type=jnp.float32) + bin_ref[...]
        h_acc = jnp.zeros((_BBLK, _D), dtype=jnp.float32)
        for bi in range(_NB):
            cur, nxt = bufs[bi % 2], bufs[(bi + 1) % 2]
            w = w_ref[...]
            wT = w[:, :, None]
            grows = grows_ref[...]
            dest = dest_ref[...]
            ptr = ptr_ref[bi]

            mem_read = jnp.sum(wT * grows, axis=1)
            h1 = jnp.tanh(emb + ctx_ref[bi] * mem_read + hid_ref[bi])
            h2 = jnp.dot(h1, Wp_ref[...], preferred_element_type=jnp.float32) + bp_ref[...]
            h2 = jnp.maximum(h2, 0.0)

            jl = jnp.sum(h2 * jgW_ref[bi, :][None, :], axis=1, keepdims=True) + jgb_ref[bi]
            p = 1.0 / (1.0 + jnp.exp(-jl))
            hard = (p > 0.5).astype(jnp.float32)
            jump = hard - p + p
            walk = jnp.mod(ptr + 1.0, float(_M))
            ptr_ref[bi] = jump * dest + (1.0 - jump) * walk
            hid_ref[bi] = h2
            h_acc = h_acc + h2

            upd_ref[...] = wT * h2[:, None, :]

            nbi = (bi + 1) % _NB
            nbase_f, nw = window_w(ptr_ref[nbi])
            w_ref[...] = nw
            nxt[...] = nbase_f.astype(jnp.int32)

            def merged(b, _):
                base_c = cur[b, 0]
                u = upd_ref[pl.ds(b, 1), :, :]
                mem_ref[pl.ds(b, 1), pl.ds(base_c + (_ROFF - _K), _WP), :] += u
                wrap_lo = base_c <= 3
                wrap_hi = base_c >= _M - 4
                start2 = jnp.where(wrap_lo, base_c + (_ROFF - _K) + _M,
                                   base_c + (_ROFF - _K) - _M)

                @pl.when(jnp.logical_or(wrap_lo, wrap_hi))
                def _():
                    mem_ref[pl.ds(b, 1), pl.ds(start2, _WP), :] += u

                base_n = nxt[b, 0]
                grows_ref[pl.ds(b, 1), :, :] = (
                    mem_ref[pl.ds(b, 1), pl.ds(base_n + (_ROFF - _K), _WP), :])
                dest_ref[pl.ds(b, 1), :] = (
                    ptrdest_ref[nbi, pl.ds(base_n, 1), :].reshape(1, 1))
                return 0

            jax.lax.fori_loop(0, _BBLK, merged, 0, unroll=8)

        out_t = jnp.dot(h_acc * (1.0 / _NB), Wout_ref[...],
                        preferred_element_type=jnp.float32) + bout_ref[...]
        out_ref[:, pl.ds(t, 1), :] = out_t[:, None, :]
        return carry

    jax.lax.fori_loop(0, _T, step, 0)


@jax.jit
def kernel(x, W_in, b_in, W_p, b_p, W_out, b_out, ptr_dest, jg_W, jg_b,
           ctx_strength, pointer_inits):
    xr = jnp.swapaxes(x, 0, 1)
    grid = (_B // _BBLK,)
    out = pl.pallas_call(
        _body,
        grid=grid,
        in_specs=[
            pl.BlockSpec((_T, _BBLK, 8), lambda i: (0, i, 0)),
            pl.BlockSpec((8, _D), lambda i: (0, 0)),
            pl.BlockSpec((1, _D), lambda i: (0, 0)),
            pl.BlockSpec((_D, _D), lambda i: (0, 0)),
            pl.BlockSpec((1, _D), lambda i: (0, 0)),
            pl.BlockSpec((_D, 8), lambda i: (0, 0)),
            pl.BlockSpec((1, 8), lambda i: (0, 0)),
            pl.BlockSpec((_NB, _M, 1), lambda i: (0, 0, 0)),
            pl.BlockSpec((_NB, _D), lambda i: (0, 0)),
            pl.BlockSpec(memory_space=pltpu.SMEM),
            pl.BlockSpec(memory_space=pltpu.SMEM),
            pl.BlockSpec((_NB, _BBLK, 1), lambda i: (0, i, 0)),
        ],
        out_specs=pl.BlockSpec((_BBLK, _T, 8), lambda i: (i, 0, 0)),
        out_shape=jax.ShapeDtypeStruct((_B, _T, 8), jnp.float32),
        scratch_shapes=[
            pltpu.VMEM((_BBLK, _MP, _D), jnp.float32),
            pltpu.VMEM((_NB, _BBLK, _D), jnp.float32),
            pltpu.VMEM((_NB, _BBLK, 1), jnp.float32),
            pltpu.VMEM((_BBLK, 1), jnp.int32),
            pltpu.VMEM((_BBLK, 1), jnp.int32),
            pltpu.VMEM((_BBLK, _WP, _D), jnp.float32),
            pltpu.VMEM((_BBLK, _WP, _D), jnp.float32),
            pltpu.VMEM((_BBLK, 1), jnp.float32),
            pltpu.VMEM((_BBLK, _WP), jnp.float32),
        ],
        compiler_params=pltpu.CompilerParams(
            dimension_semantics=("arbitrary",),
            disable_bounds_checks=True,
        ),
    )(xr, W_in, b_in.reshape(1, _D), W_p, b_p.reshape(1, _D), W_out,
      b_out.reshape(1, 8), ptr_dest[..., None], jg_W, jg_b, ctx_strength,
      pointer_inits[..., None])
    return out


# speedup vs baseline: 3.4721x; 1.0164x over previous
import functools

import jax
import jax.numpy as jnp
from jax.experimental import pallas as pl
from jax.experimental.pallas import tpu as pltpu

_M = 2048
_D = 64
_NB = 4
_K = 2
_W = 2 * _K + 1
_WP = 8
_TEMP = 8.0
_T = 16
_B = 256
_BBLK = 32
_MP = _M + 24
_ROFF = 8
_HALF = _M / 2.0


def _body(x_ref, Win_ref, bin_ref, Wp_ref, bp_ref, Wout_ref, bout_ref,
          ptrdest_ref, jgW_ref, jgb_ref, ctx_ref, ptr0_ref,
          out_ref,
          mem_ref, hid_ref, ptr_ref, basei_a, basei_b, grows_ref, upd_ref,
          dest_ref, w_ref, wt_ref):
    mem_ref[...] = jnp.zeros_like(mem_ref)
    hid_ref[...] = jnp.zeros_like(hid_ref)
    upd_ref[...] = jnp.zeros_like(upd_ref)
    grows_ref[...] = jnp.zeros_like(grows_ref)
    ptr_ref[...] = ptr0_ref[...]
    bufs = (basei_a, basei_b)

    def dest_onehot(base_f, nbi):
        iom = jax.lax.broadcasted_iota(jnp.int32, (_BBLK, _M), 1)
        oh = (iom == base_f.astype(jnp.int32)).astype(jnp.float32)
        dall = jnp.dot(oh, ptrdest_ref[...],
                       preferred_element_type=jnp.float32)
        return dall[:, nbi:nbi + 1]

    def window_w(ptr):
        base_f = jnp.clip(jnp.floor(ptr), 0.0, _M - 1.0)
        io = jax.lax.broadcasted_iota(jnp.int32, (_BBLK, _WP), 1).astype(jnp.float32)
        idx_f = jnp.mod(base_f + (io - _K), float(_M))
        delta = jnp.mod(idx_f - ptr + _HALF, float(_M)) - _HALF
        z = -(delta * delta) / _TEMP
        live = io < float(_W)
        z = jnp.where(live, z, -jnp.inf)
        z = z - jnp.max(z, axis=1, keepdims=True)
        e = jnp.exp(z)
        e = jnp.where(live, e, 0.0)
        return base_f, e / jnp.sum(e, axis=1, keepdims=True)

    base_f0, w0 = window_w(ptr_ref[0])
    w_ref[...] = w0
    basei_a[...] = base_f0.astype(jnp.int32)
    dest_ref[...] = dest_onehot(base_f0, 0)

    def step(t, carry):
        xt = x_ref[pl.ds(t, 1), :, :].reshape(_BBLK, 8)
        emb = jnp.dot(xt, Win_ref[...], preferred_element_type=jnp.float32) + bin_ref[...]
        h_acc = jnp.zeros((_BBLK, _D), dtype=jnp.float32)
        for bi in range(_NB):
            cur, nxt = bufs[bi % 2], bufs[(bi + 1) % 2]
            w = w_ref[...]
            wt_ref[...] = jnp.broadcast_to(w[:, :, None], (_BBLK, _WP, _D))
            wT = wt_ref[...]
            grows = grows_ref[...]
            dest = dest_ref[...]
            ptr = ptr_ref[bi]

            mem_read = jnp.sum(wT * grows, axis=1)
            h1 = jnp.tanh(emb + ctx_ref[bi] * mem_read + hid_ref[bi])
            h2 = jnp.dot(h1, Wp_ref[...], preferred_element_type=jnp.float32) + bp_ref[...]
            h2 = jnp.maximum(h2, 0.0)

            jl = jnp.sum(h2 * jgW_ref[bi, :][None, :], axis=1, keepdims=True) + jgb_ref[bi]
            p = 1.0 / (1.0 + jnp.exp(-jl))
            hard = (p > 0.5).astype(jnp.float32)
            jump = hard - p + p
            walk = jnp.mod(ptr + 1.0, float(_M))
            ptr_ref[bi] = jump * dest + (1.0 - jump) * walk
            hid_ref[bi] = h2
            h_acc = h_acc + h2

            upd_ref[...] = wT * h2[:, None, :]

            nbi = (bi + 1) % _NB
            nbase_f, nw = window_w(ptr_ref[nbi])
            w_ref[...] = nw
            nxt[...] = nbase_f.astype(jnp.int32)
            dest_ref[...] = dest_onehot(nbase_f, nbi)

            def merged(b, _):
                base_c = cur[b, 0]
                u = upd_ref[pl.ds(b, 1), :, :]
                mem_ref[pl.ds(b, 1), pl.ds(base_c + (_ROFF - _K), _WP), :] += u
                wrap_lo = base_c <= 3
                wrap_hi = base_c >= _M - 4
                start2 = jnp.where(wrap_lo, base_c + (_ROFF - _K) + _M,
                                   base_c + (_ROFF - _K) - _M)

                @pl.when(jnp.logical_or(wrap_lo, wrap_hi))
                def _():
                    mem_ref[pl.ds(b, 1), pl.ds(start2, _WP), :] += u

                base_n = nxt[b, 0]
                grows_ref[pl.ds(b, 1), :, :] = (
                    mem_ref[pl.ds(b, 1), pl.ds(base_n + (_ROFF - _K), _WP), :])
                return 0

            jax.lax.fori_loop(0, _BBLK, merged, 0, unroll=16)

        out_t = jnp.dot(h_acc * (1.0 / _NB), Wout_ref[...],
                        preferred_element_type=jnp.float32) + bout_ref[...]
        out_ref[:, pl.ds(t, 1), :] = out_t[:, None, :]
        return carry

    jax.lax.fori_loop(0, _T, step, 0)


@jax.jit
def kernel(x, W_in, b_in, W_p, b_p, W_out, b_out, ptr_dest, jg_W, jg_b,
           ctx_strength, pointer_inits):
    xr = jnp.swapaxes(x, 0, 1)
    grid = (_B // _BBLK,)
    out = pl.pallas_call(
        _body,
        grid=grid,
        in_specs=[
            pl.BlockSpec((_T, _BBLK, 8), lambda i: (0, i, 0)),
            pl.BlockSpec((8, _D), lambda i: (0, 0)),
            pl.BlockSpec((1, _D), lambda i: (0, 0)),
            pl.BlockSpec((_D, _D), lambda i: (0, 0)),
            pl.BlockSpec((1, _D), lambda i: (0, 0)),
            pl.BlockSpec((_D, 8), lambda i: (0, 0)),
            pl.BlockSpec((1, 8), lambda i: (0, 0)),
            pl.BlockSpec((_M, _NB), lambda i: (0, 0)),
            pl.BlockSpec((_NB, _D), lambda i: (0, 0)),
            pl.BlockSpec(memory_space=pltpu.SMEM),
            pl.BlockSpec(memory_space=pltpu.SMEM),
            pl.BlockSpec((_NB, _BBLK, 1), lambda i: (0, i, 0)),
        ],
        out_specs=pl.BlockSpec((_BBLK, _T, 8), lambda i: (i, 0, 0)),
        out_shape=jax.ShapeDtypeStruct((_B, _T, 8), jnp.float32),
        scratch_shapes=[
            pltpu.VMEM((_BBLK, _MP, _D), jnp.float32),
            pltpu.VMEM((_NB, _BBLK, _D), jnp.float32),
            pltpu.VMEM((_NB, _BBLK, 1), jnp.float32),
            pltpu.VMEM((_BBLK, 1), jnp.int32),
            pltpu.VMEM((_BBLK, 1), jnp.int32),
            pltpu.VMEM((_BBLK, _WP, _D), jnp.float32),
            pltpu.VMEM((_BBLK, _WP, _D), jnp.float32),
            pltpu.VMEM((_BBLK, 1), jnp.float32),
            pltpu.VMEM((_BBLK, _WP), jnp.float32),
            pltpu.VMEM((_BBLK, _WP, _D), jnp.float32),
        ],
        compiler_params=pltpu.CompilerParams(
            dimension_semantics=("arbitrary",),
            disable_bounds_checks=True,
        ),
    )(xr, W_in, b_in.reshape(1, _D), W_p, b_p.reshape(1, _D), W_out,
      b_out.reshape(1, 8), jnp.swapaxes(ptr_dest, 0, 1), jg_W, jg_b,
      ctx_strength, pointer_inits[..., None])
    return out


# speedup vs baseline: 3.4759x; 1.0011x over previous
import functools

import jax
import jax.numpy as jnp
from jax.experimental import pallas as pl
from jax.experimental.pallas import tpu as pltpu

_M = 2048
_D = 64
_NB = 4
_K = 2
_W = 2 * _K + 1
_WP = 8
_TEMP = 8.0
_T = 16
_B = 256
_BBLK = 32
_NC = 4
_MP = _M + 24
_ROFF = 8
_HALF = _M / 2.0


def _body(x_ref, Win_ref, bin_ref, Wp_ref, bp_ref, Wout_ref, bout_ref,
          ptrdest_ref, jgW_ref, jgb_ref, ctx_ref, ptr0_ref,
          out_ref,
          mem0, mem1, mem2, mem3, hid_ref, ptr_ref, basei_a, basei_b,
          grows_ref, upd_ref, dest_ref, w_ref, wt_ref):
    mems = (mem0, mem1, mem2, mem3)
    for m in mems:
        m[...] = jnp.zeros_like(m)
    hid_ref[...] = jnp.zeros_like(hid_ref)
    upd_ref[...] = jnp.zeros_like(upd_ref)
    grows_ref[...] = jnp.zeros_like(grows_ref)
    ptr_ref[...] = ptr0_ref[...]
    bufs = (basei_a, basei_b)

    def dest_onehot(base_f, nbi):
        iom = jax.lax.broadcasted_iota(jnp.int32, (_BBLK, _M), 1)
        oh = (iom == base_f.astype(jnp.int32)).astype(jnp.float32)
        dall = jnp.dot(oh, ptrdest_ref[...],
                       preferred_element_type=jnp.float32)
        return dall[:, nbi:nbi + 1]

    def window_w(ptr):
        base_f = jnp.clip(jnp.floor(ptr), 0.0, _M - 1.0)
        io = jax.lax.broadcasted_iota(jnp.int32, (_BBLK, _WP), 1).astype(jnp.float32)
        idx_f = jnp.mod(base_f + (io - _K), float(_M))
        delta = jnp.mod(idx_f - ptr + _HALF, float(_M)) - _HALF
        z = -(delta * delta) / _TEMP
        live = io < float(_W)
        z = jnp.where(live, z, -jnp.inf)
        z = z - jnp.max(z, axis=1, keepdims=True)
        e = jnp.exp(z)
        e = jnp.where(live, e, 0.0)
        return base_f, e / jnp.sum(e, axis=1, keepdims=True)

    base_f0, w0 = window_w(ptr_ref[0])
    w_ref[...] = w0
    basei_a[...] = base_f0.astype(jnp.int32)
    dest_ref[...] = dest_onehot(base_f0, 0)

    def step(t, carry):
        xt = x_ref[pl.ds(t, 1), :, :].reshape(_BBLK, 8)
        emb = jnp.dot(xt, Win_ref[...], preferred_element_type=jnp.float32) + bin_ref[...]
        h_acc = jnp.zeros((_BBLK, _D), dtype=jnp.float32)
        for bi in range(_NB):
            cur, nxt = bufs[bi % 2], bufs[(bi + 1) % 2]
            w = w_ref[...]
            wt_ref[...] = jnp.broadcast_to(w[:, :, None], (_BBLK, _WP, _D))
            wT = wt_ref[...]
            grows = grows_ref[...]
            dest = dest_ref[...]
            ptr = ptr_ref[bi]

            mem_read = jnp.sum(wT * grows, axis=1)
            h1 = jnp.tanh(emb + ctx_ref[bi] * mem_read + hid_ref[bi])
            h2 = jnp.dot(h1, Wp_ref[...], preferred_element_type=jnp.float32) + bp_ref[...]
            h2 = jnp.maximum(h2, 0.0)

            jl = jnp.sum(h2 * jgW_ref[bi, :][None, :], axis=1, keepdims=True) + jgb_ref[bi]
            p = 1.0 / (1.0 + jnp.exp(-jl))
            hard = (p > 0.5).astype(jnp.float32)
            jump = hard - p + p
            walk = jnp.mod(ptr + 1.0, float(_M))
            ptr_ref[bi] = jump * dest + (1.0 - jump) * walk
            hid_ref[bi] = h2
            h_acc = h_acc + h2

            upd_ref[...] = wT * h2[:, None, :]

            nbi = (bi + 1) % _NB
            nbase_f, nw = window_w(ptr_ref[nbi])
            w_ref[...] = nw
            nxt[...] = nbase_f.astype(jnp.int32)
            dest_ref[...] = dest_onehot(nbase_f, nbi)

            def merged(g, _):
                for c in range(_NC):
                    b = c * (_BBLK // _NC) + g
                    mref = mems[c]
                    base_c = cur[b, 0]
                    u = upd_ref[pl.ds(b, 1), :, :]
                    mref[pl.ds(g, 1), pl.ds(base_c + (_ROFF - _K), _WP), :] += u
                    wrap_lo = base_c <= 3
                    wrap_hi = base_c >= _M - 4
                    start2 = jnp.where(wrap_lo, base_c + (_ROFF - _K) + _M,
                                       base_c + (_ROFF - _K) - _M)

                    @pl.when(jnp.logical_or(wrap_lo, wrap_hi))
                    def _():
                        mref[pl.ds(g, 1), pl.ds(start2, _WP), :] += u

                    base_n = nxt[b, 0]
                    grows_ref[pl.ds(b, 1), :, :] = (
                        mref[pl.ds(g, 1), pl.ds(base_n + (_ROFF - _K), _WP), :])
                return 0

            jax.lax.fori_loop(0, _BBLK // _NC, merged, 0, unroll=4)

        out_t = jnp.dot(h_acc * (1.0 / _NB), Wout_ref[...],
                        preferred_element_type=jnp.float32) + bout_ref[...]
        out_ref[:, pl.ds(t, 1), :] = out_t[:, None, :]
        return carry

    jax.lax.fori_loop(0, _T, step, 0)


@jax.jit
def kernel(x, W_in, b_in, W_p, b_p, W_out, b_out, ptr_dest, jg_W, jg_b,
           ctx_strength, pointer_inits):
    xr = jnp.swapaxes(x, 0, 1)
    grid = (_B // _BBLK,)
    out = pl.pallas_call(
        _body,
        grid=grid,
        in_specs=[
            pl.BlockSpec((_T, _BBLK, 8), lambda i: (0, i, 0)),
            pl.BlockSpec((8, _D), lambda i: (0, 0)),
            pl.BlockSpec((1, _D), lambda i: (0, 0)),
            pl.BlockSpec((_D, _D), lambda i: (0, 0)),
            pl.BlockSpec((1, _D), lambda i: (0, 0)),
            pl.BlockSpec((_D, 8), lambda i: (0, 0)),
            pl.BlockSpec((1, 8), lambda i: (0, 0)),
            pl.BlockSpec((_M, _NB), lambda i: (0, 0)),
            pl.BlockSpec((_NB, _D), lambda i: (0, 0)),
            pl.BlockSpec(memory_space=pltpu.SMEM),
            pl.BlockSpec(memory_space=pltpu.SMEM),
            pl.BlockSpec((_NB, _BBLK, 1), lambda i: (0, i, 0)),
        ],
        out_specs=pl.BlockSpec((_BBLK, _T, 8), lambda i: (i, 0, 0)),
        out_shape=jax.ShapeDtypeStruct((_B, _T, 8), jnp.float32),
        scratch_shapes=[
            pltpu.VMEM((_BBLK // _NC, _MP, _D), jnp.float32),
            pltpu.VMEM((_BBLK // _NC, _MP, _D), jnp.float32),
            pltpu.VMEM((_BBLK // _NC, _MP, _D), jnp.float32),
            pltpu.VMEM((_BBLK // _NC, _MP, _D), jnp.float32),
            pltpu.VMEM((_NB, _BBLK, _D), jnp.float32),
            pltpu.VMEM((_NB, _BBLK, 1), jnp.float32),
            pltpu.VMEM((_BBLK, 1), jnp.int32),
            pltpu.VMEM((_BBLK, 1), jnp.int32),
            pltpu.VMEM((_BBLK, _WP, _D), jnp.float32),
            pltpu.VMEM((_BBLK, _WP, _D), jnp.float32),
            pltpu.VMEM((_BBLK, 1), jnp.float32),
            pltpu.VMEM((_BBLK, _WP), jnp.float32),
            pltpu.VMEM((_BBLK, _WP, _D), jnp.float32),
        ],
        compiler_params=pltpu.CompilerParams(
            dimension_semantics=("arbitrary",),
            disable_bounds_checks=True,
        ),
    )(xr, W_in, b_in.reshape(1, _D), W_p, b_p.reshape(1, _D), W_out,
      b_out.reshape(1, 8), jnp.swapaxes(ptr_dest, 0, 1), jg_W, jg_b,
      ctx_strength, pointer_inits[..., None])
    return out


# speedup vs baseline: 5.2180x; 1.5012x over previous
import functools

import jax
import jax.numpy as jnp
from jax.experimental import pallas as pl
from jax.experimental.pallas import tpu as pltpu

_M = 2048
_D = 64
_NB = 4
_K = 2
_W = 2 * _K + 1
_WP = 8
_TEMP = 8.0
_T = 16
_B = 256
_BBLK = 32
_NC = 4
_MP = _M + 24
_ROFF = 8
_HALF = _M / 2.0


def _body(x_ref, Win_ref, bin_ref, Wp_ref, bp_ref, Wout_ref, bout_ref,
          ptrdest_ref, jgW_ref, jgb_ref, ctx_ref, ptr0_ref,
          out_ref,
          mem0, mem1, mem2, mem3, hid_ref, ptr_ref, basei_a, basei_b,
          smem_a, smem_b, dma_sem, grows_ref, upd_ref, dest_ref, w_ref,
          wt_ref):
    mems = (mem0, mem1, mem2, mem3)
    for m in mems:
        m[...] = jnp.zeros_like(m)
    hid_ref[...] = jnp.zeros_like(hid_ref)
    upd_ref[...] = jnp.zeros_like(upd_ref)
    grows_ref[...] = jnp.zeros_like(grows_ref)
    ptr_ref[...] = ptr0_ref[...]
    bufs = (basei_a, basei_b)

    def dest_onehot(base_f, nbi):
        iom = jax.lax.broadcasted_iota(jnp.int32, (_BBLK, _M), 1)
        oh = (iom == base_f.astype(jnp.int32)).astype(jnp.float32)
        dall = jnp.dot(oh, ptrdest_ref[...],
                       preferred_element_type=jnp.float32)
        return dall[:, nbi:nbi + 1]

    def window_w(ptr):
        base_f = jnp.clip(jnp.floor(ptr), 0.0, _M - 1.0)
        io = jax.lax.broadcasted_iota(jnp.int32, (_BBLK, _WP), 1).astype(jnp.float32)
        idx_f = jnp.mod(base_f + (io - _K), float(_M))
        delta = jnp.mod(idx_f - ptr + _HALF, float(_M)) - _HALF
        z = -(delta * delta) / _TEMP
        live = io < float(_W)
        z = jnp.where(live, z, -jnp.inf)
        z = z - jnp.max(z, axis=1, keepdims=True)
        e = jnp.exp(z)
        e = jnp.where(live, e, 0.0)
        return base_f, e / jnp.sum(e, axis=1, keepdims=True)

    base_f0, w0 = window_w(ptr_ref[0])
    w_ref[...] = w0
    basei_a[...] = base_f0.astype(jnp.int32)
    pltpu.make_async_copy(basei_a, smem_a, dma_sem).start()
    dest_ref[...] = dest_onehot(base_f0, 0)
    pltpu.make_async_copy(basei_a, smem_a, dma_sem).wait()
    sbufs = (smem_a, smem_b)

    def step(t, carry):
        xt = x_ref[pl.ds(t, 1), :, :].reshape(_BBLK, 8)
        emb = jnp.dot(xt, Win_ref[...], preferred_element_type=jnp.float32) + bin_ref[...]
        h_acc = jnp.zeros((_BBLK, _D), dtype=jnp.float32)
        for bi in range(_NB):
            cur, nxt = sbufs[bi % 2], sbufs[(bi + 1) % 2]
            nxt_v = bufs[(bi + 1) % 2]
            w = w_ref[...]
            wt_ref[...] = jnp.broadcast_to(w[:, :, None], (_BBLK, _WP, _D))
            wT = wt_ref[...]
            grows = grows_ref[...]
            dest = dest_ref[...]
            ptr = ptr_ref[bi]

            mem_read = jnp.sum(wT * grows, axis=1)
            h1 = jnp.tanh(emb + ctx_ref[bi] * mem_read + hid_ref[bi])
            h2 = jnp.dot(h1, Wp_ref[...], preferred_element_type=jnp.float32) + bp_ref[...]
            h2 = jnp.maximum(h2, 0.0)

            jl = jnp.sum(h2 * jgW_ref[bi, :][None, :], axis=1, keepdims=True) + jgb_ref[bi]
            p = 1.0 / (1.0 + jnp.exp(-jl))
            hard = (p > 0.5).astype(jnp.float32)
            jump = hard - p + p
            walk = jnp.mod(ptr + 1.0, float(_M))
            ptr_ref[bi] = jump * dest + (1.0 - jump) * walk
            hid_ref[bi] = h2
            h_acc = h_acc + h2

            upd_ref[...] = wT * h2[:, None, :]

            nbi = (bi + 1) % _NB
            nbase_f, nw = window_w(ptr_ref[nbi])
            w_ref[...] = nw
            nxt_v[...] = nbase_f.astype(jnp.int32)
            pltpu.make_async_copy(nxt_v, nxt, dma_sem).start()
            dest_ref[...] = dest_onehot(nbase_f, nbi)
            pltpu.make_async_copy(nxt_v, nxt, dma_sem).wait()

            def merged(g, _):
                for c in range(_NC):
                    b = c * (_BBLK // _NC) + g
                    mref = mems[c]
                    base_c = cur[b, 0]
                    u = upd_ref[pl.ds(b, 1), :, :]
                    mref[pl.ds(g, 1), pl.ds(base_c + (_ROFF - _K), _WP), :] += u
                    wrap_lo = base_c <= 3
                    wrap_hi = base_c >= _M - 4
                    start2 = jnp.where(wrap_lo, base_c + (_ROFF - _K) + _M,
                                       base_c + (_ROFF - _K) - _M)

                    @pl.when(jnp.logical_or(wrap_lo, wrap_hi))
                    def _():
                        mref[pl.ds(g, 1), pl.ds(start2, _WP), :] += u

                    base_n = nxt[b, 0]
                    grows_ref[pl.ds(b, 1), :, :] = (
                        mref[pl.ds(g, 1), pl.ds(base_n + (_ROFF - _K), _WP), :])
                return 0

            jax.lax.fori_loop(0, _BBLK // _NC, merged, 0, unroll=4)

        out_t = jnp.dot(h_acc * (1.0 / _NB), Wout_ref[...],
                        preferred_element_type=jnp.float32) + bout_ref[...]
        out_ref[:, pl.ds(t, 1), :] = out_t[:, None, :]
        return carry

    jax.lax.fori_loop(0, _T, step, 0)


@jax.jit
def kernel(x, W_in, b_in, W_p, b_p, W_out, b_out, ptr_dest, jg_W, jg_b,
           ctx_strength, pointer_inits):
    xr = jnp.swapaxes(x, 0, 1)
    grid = (_B // _BBLK,)
    out = pl.pallas_call(
        _body,
        grid=grid,
        in_specs=[
            pl.BlockSpec((_T, _BBLK, 8), lambda i: (0, i, 0)),
            pl.BlockSpec((8, _D), lambda i: (0, 0)),
            pl.BlockSpec((1, _D), lambda i: (0, 0)),
            pl.BlockSpec((_D, _D), lambda i: (0, 0)),
            pl.BlockSpec((1, _D), lambda i: (0, 0)),
            pl.BlockSpec((_D, 8), lambda i: (0, 0)),
            pl.BlockSpec((1, 8), lambda i: (0, 0)),
            pl.BlockSpec((_M, _NB), lambda i: (0, 0)),
            pl.BlockSpec((_NB, _D), lambda i: (0, 0)),
            pl.BlockSpec(memory_space=pltpu.SMEM),
            pl.BlockSpec(memory_space=pltpu.SMEM),
            pl.BlockSpec((_NB, _BBLK, 1), lambda i: (0, i, 0)),
        ],
        out_specs=pl.BlockSpec((_BBLK, _T, 8), lambda i: (i, 0, 0)),
        out_shape=jax.ShapeDtypeStruct((_B, _T, 8), jnp.float32),
        scratch_shapes=[
            pltpu.VMEM((_BBLK // _NC, _MP, _D), jnp.float32),
            pltpu.VMEM((_BBLK // _NC, _MP, _D), jnp.float32),
            pltpu.VMEM((_BBLK // _NC, _MP, _D), jnp.float32),
            pltpu.VMEM((_BBLK // _NC, _MP, _D), jnp.float32),
            pltpu.VMEM((_NB, _BBLK, _D), jnp.float32),
            pltpu.VMEM((_NB, _BBLK, 1), jnp.float32),
            pltpu.VMEM((_BBLK, 1), jnp.int32),
            pltpu.VMEM((_BBLK, 1), jnp.int32),
            pltpu.SMEM((_BBLK, 1), jnp.int32),
            pltpu.SMEM((_BBLK, 1), jnp.int32),
            pltpu.SemaphoreType.DMA,
            pltpu.VMEM((_BBLK, _WP, _D), jnp.float32),
            pltpu.VMEM((_BBLK, _WP, _D), jnp.float32),
            pltpu.VMEM((_BBLK, 1), jnp.float32),
            pltpu.VMEM((_BBLK, _WP), jnp.float32),
            pltpu.VMEM((_BBLK, _WP, _D), jnp.float32),
        ],
        compiler_params=pltpu.CompilerParams(
            dimension_semantics=("arbitrary",),
            disable_bounds_checks=True,
        ),
    )(xr, W_in, b_in.reshape(1, _D), W_p, b_p.reshape(1, _D), W_out,
      b_out.reshape(1, 8), jnp.swapaxes(ptr_dest, 0, 1), jg_W, jg_b,
      ctx_strength, pointer_inits[..., None])
    return out
